# baseline (speedup 1.0000x reference)
import functools

import jax
import jax.numpy as jnp
from jax import lax
from jax.experimental import pallas as pl
from jax.experimental.pallas import tpu as pltpu
from jax.experimental.pallas import tpu_sc as plsc

_NUM_CORES = 2
_NUM_SUBCORES = 16
_NUM_WORKERS = _NUM_CORES * _NUM_SUBCORES
_CHUNK = 128
_NSLOT = 3
_MSLOT = 3
_AHEAD = 2
_LANES = 16


def _bcast_lane(vec, j):
    idx = jnp.full((_LANES, 1), j, dtype=jnp.int32)
    dn = lax.GatherDimensionNumbers(
        offset_dims=(), collapsed_slice_dims=(0,), start_index_map=(0,)
    )
    return lax.gather(
        vec, idx, dn, slice_sizes=(1,),
        mode=lax.GatherScatterMode.PROMISE_IN_BOUNDS,
    )


def kernel(token_ids, weight):
    batch, seq = token_ids.shape
    nrows, dim = weight.shape
    total = batch * seq
    dim2 = 2 * dim
    crows = _CHUNK // 2

    per_worker = total // _NUM_WORKERS
    chunks = per_worker // _CHUNK
    out_pw = per_worker // 2

    ids = token_ids.astype(jnp.int32).reshape(total)
    pairs = ids >> 1
    hoffs = (ids & 1) * dim
    table2 = weight.reshape(nrows // 2, dim2)

    mesh = plsc.VectorSubcoreMesh(core_axis_name="c", subcore_axis_name="s")

    @functools.partial(
        pl.kernel,
        mesh=mesh,
        out_type=jax.ShapeDtypeStruct((total // 2, dim2), weight.dtype),
        scratch_types=[
            pltpu.VMEM((per_worker,), jnp.int32),
            pltpu.VMEM((per_worker + _LANES,), jnp.int32),
            pltpu.VMEM((_NSLOT, _CHUNK, dim2), jnp.float32),
            pltpu.VMEM((_MSLOT, crows, dim2), jnp.float32),
            pltpu.SemaphoreType.DMA((_NSLOT,)),
            pltpu.SemaphoreType.DMA((_MSLOT,)),
        ],
        compiler_params=pltpu.CompilerParams(use_tc_tiling_on_sc=True),
    )
    def gather_kernel(
        table_hbm, pair_hbm, hoff_hbm, out_hbm, pair_v, hoff_v, rows_v, comp_v,
        gsem, osem,
    ):
        wid = lax.axis_index("s") * _NUM_CORES + lax.axis_index("c")
        base = wid * per_worker
        obase = wid * out_pw
        pltpu.sync_copy(pair_hbm.at[pl.ds(base, per_worker)], pair_v)
        pltpu.sync_copy(
            hoff_hbm.at[pl.ds(base, per_worker)],
            hoff_v.at[pl.ds(0, per_worker)],
        )

        iota = lax.iota(jnp.int32, _LANES)
        gather_d = {}
        out_d = {}

        def start_gather(c):
            slot = c % _NSLOT
            gather_d[c] = pltpu.async_copy(
                table_hbm.at[pair_v.at[pl.ds(c * _CHUNK, _CHUNK)]],
                rows_v.at[slot],
                gsem.at[slot],
            )

        def compact(c):
            gslot = c % _NSLOT
            mslot = c % _MSLOT
            src = rows_v.at[gslot]
            dst = comp_v.at[mslot]

            def body(p, _):
                h = hoff_v[pl.ds(c * _CHUNK + p, _LANES)][0]
                for k in range(dim // _LANES):
                    vals = src[p, pl.ds(h + k * _LANES, _LANES)]
                    dst[p // 2, pl.ds((p % 2) * dim + k * _LANES, _LANES)] = vals
                return 0

            lax.fori_loop(0, _CHUNK, body, 0)

        def start_out(c):
            mslot = c % _MSLOT
            out_d[c] = pltpu.async_copy(
                comp_v.at[mslot],
                out_hbm.at[pl.ds(obase + c * crows, crows)],
                osem.at[mslot],
            )

        for j in range(_AHEAD):
            start_gather(j)
        for c in range(chunks):
            j = c + _AHEAD
            if j < chunks:
                start_gather(j)
            gather_d[c].wait()
            if c >= _MSLOT:
                out_d[c - _MSLOT].wait()
            compact(c)
            start_out(c)
        for c in range(max(0, chunks - _MSLOT), chunks):
            out_d[c].wait()

    out2 = gather_kernel(table2, pairs, hoffs)
    return out2.reshape(batch, seq, dim)


# speedup vs baseline: 1.2018x; 1.2018x over previous
import functools

import jax
import jax.numpy as jnp
from jax import lax
from jax.experimental import pallas as pl
from jax.experimental.pallas import tpu as pltpu
from jax.experimental.pallas import tpu_sc as plsc

_NUM_CORES = 2
_NUM_SUBCORES = 16
_NUM_WORKERS = _NUM_CORES * _NUM_SUBCORES
_CHUNK = 128
_NSLOT = 8
_AHEAD = 4


def kernel(token_ids, weight):
    batch, seq = token_ids.shape
    dim = weight.shape[1]
    total = batch * seq

    per_worker = total // _NUM_WORKERS
    chunks = per_worker // _CHUNK

    mesh = plsc.VectorSubcoreMesh(core_axis_name="c", subcore_axis_name="s")

    @functools.partial(
        pl.kernel,
        mesh=mesh,
        out_type=jax.ShapeDtypeStruct((total, dim), weight.dtype),
        scratch_types=[
            pltpu.VMEM((per_worker,), jnp.int32),
            pltpu.VMEM((_NSLOT, _CHUNK, dim), jnp.float32),
            pltpu.SemaphoreType.DMA((_NSLOT,)),
            pltpu.SemaphoreType.DMA((_NSLOT,)),
        ],
        compiler_params=pltpu.CompilerParams(use_tc_tiling_on_sc=False),
    )
    def gather_kernel(table_hbm, idx_hbm, out_hbm, idx_v, rows_v, gsem, osem):
        wid = lax.axis_index("s") * _NUM_CORES + lax.axis_index("c")
        base = wid * per_worker
        pltpu.sync_copy(idx_hbm.at[pl.ds(base, per_worker)], idx_v)

        gather_d = {}
        out_d = {}

        def start_gather(c):
            slot = c % _NSLOT
            gather_d[c] = pltpu.async_copy(
                table_hbm.at[idx_v.at[pl.ds(c * _CHUNK, _CHUNK)]],
                rows_v.at[slot],
                gsem.at[slot],
            )

        def start_out(c):
            slot = c % _NSLOT
            out_d[c] = pltpu.async_copy(
                rows_v.at[slot],
                out_hbm.at[pl.ds(base + c * _CHUNK, _CHUNK)],
                osem.at[slot],
            )

        for j in range(_AHEAD):
            start_gather(j)
        for c in range(chunks):
            j = c + _AHEAD
            if j < chunks:
                if j >= _NSLOT:
                    out_d[j - _NSLOT].wait()
                start_gather(j)
            gather_d[c].wait()
            start_out(c)
        for c in range(max(0, chunks - _NSLOT), chunks):
            out_d[c].wait()

    flat_ids = token_ids.astype(jnp.int32).reshape(total)
    out = gather_kernel(weight, flat_ids)
    return out.reshape(batch, seq, dim)
